# baseline (speedup 1.0000x reference)
import functools

import jax
import jax.numpy as jnp
from jax import lax
from jax.experimental import pallas as pl
from jax.experimental.pallas import tpu as pltpu
from jax.experimental.pallas import tpu_sc as plsc

NUM_CLASSES = 1000000
COND_SIZE = 64
BATCH = 16384

NUM_CORES = 2
NUM_SUBCORES = 16
NUM_WORKERS = NUM_CORES * NUM_SUBCORES
B_PER_W = BATCH // NUM_WORKERS


def _make_gather():
    mesh = plsc.VectorSubcoreMesh(core_axis_name="c", subcore_axis_name="s")

    @functools.partial(
        pl.kernel,
        mesh=mesh,
        out_type=jax.ShapeDtypeStruct((BATCH, COND_SIZE), jnp.float32),
        scratch_types=[
            pltpu.VMEM((B_PER_W,), jnp.int32),
            pltpu.VMEM((B_PER_W, COND_SIZE), jnp.float32),
            pltpu.SemaphoreType.DMA,
        ],
    )
    def gather_kernel(labels_hbm, table_hbm, out_hbm, idx_v, rows_v, sem):
        wid = lax.axis_index("s") * NUM_CORES + lax.axis_index("c")
        base = wid * B_PER_W
        pltpu.sync_copy(labels_hbm.at[pl.ds(base, B_PER_W)], idx_v)

        def fire(g, _):
            labs = idx_v[pl.ds(g * 16, 16)]
            for j in range(16):
                pltpu.async_copy(
                    table_hbm.at[pl.ds(labs[j], 1)],
                    rows_v.at[pl.ds(g * 16 + j, 1)],
                    sem,
                )
            return 0

        lax.fori_loop(0, B_PER_W // 16, fire, 0)
        pltpu.make_async_copy(
            table_hbm.at[pl.ds(0, B_PER_W)], rows_v, sem
        ).wait()
        pltpu.sync_copy(rows_v, out_hbm.at[pl.ds(base, B_PER_W)])

    return gather_kernel


_gather = _make_gather()


def kernel(labels, embedding_table):
    return _gather(labels.astype(jnp.int32), embedding_table)
